# baseline (speedup 1.0000x reference)
import functools

import jax
import jax.numpy as jnp
from jax import lax
from jax.experimental import pallas as pl
from jax.experimental.pallas import tpu as pltpu
from jax.experimental.pallas import tpu_sc as plsc

NUM_NODES = 100000
NUM_SEGMENTS = 512
D = 128

NC = 2
NS = 16
NW = NC * NS
L = 16
DJ = D // L

G = 16
NGROUPS = NUM_NODES // G
GQ, GR = divmod(NGROUPS, NW)
MAX_GROUPS = GQ + 1
IDX_MAIN = GQ * G
IDX_PAD = MAX_GROUPS * G

R = 3
CH = 64
CH_G = CH // G
NCH = (MAX_GROUPS + CH_G - 1) // CH_G
SUMW = NUM_SEGMENTS * D
CNTW = NUM_SEGMENTS * L


def _sc_partial(v_flat, batch):
  mesh = plsc.VectorSubcoreMesh(core_axis_name="c", subcore_axis_name="s")

  @functools.partial(
      pl.kernel,
      out_type=[
          jax.ShapeDtypeStruct((NW * SUMW,), jnp.float32),
          jax.ShapeDtypeStruct((NW, CNTW), jnp.float32),
      ],
      mesh=mesh,
      scratch_types=[
          pltpu.VMEM((IDX_PAD,), jnp.int32),
          pltpu.VMEM((R * CH * D,), jnp.float32),
          pltpu.VMEM((SUMW,), jnp.float32),
          pltpu.VMEM((CNTW,), jnp.float32),
          pltpu.SemaphoreType.DMA((R,)),
          pltpu.SemaphoreType.DMA((2,)),
      ],
  )
  def k(v_hbm, b_hbm, psum_hbm, pcnt_hbm, idx_v, ring, loc_s, loc_c, sems,
        isems):
    cid = lax.axis_index("c")
    sid = lax.axis_index("s")
    wid = sid * NC + cid

    base_group = wid * GQ + jnp.minimum(wid, GR)
    ngroups = GQ + jnp.where(wid < GR, 1, 0)
    row_base = base_group * G
    nrows = ngroups * G

    zeros = jnp.zeros((L,), jnp.float32)
    ones = jnp.ones((L,), jnp.float32)
    sixteens = jnp.full((L,), 16.0)

    def issue(c, slot):
      full = (c + 1) * CH <= nrows
      part = nrows - c * CH == CH - G

      @pl.when(full)
      def _():
        pltpu.async_copy(
            v_hbm.at[pl.ds((row_base + c * CH) * D, CH * D)],
            ring.at[pl.ds(slot * CH * D, CH * D)], sems.at[slot])

      @pl.when(part)
      def _():
        pltpu.async_copy(
            v_hbm.at[pl.ds((row_base + c * CH) * D, (CH - G) * D)],
            ring.at[pl.ds(slot * CH * D, (CH - G) * D)], sems.at[slot])

    def wait_chunk(c, slot):
      full = (c + 1) * CH <= nrows
      part = nrows - c * CH == CH - G

      @pl.when(full)
      def _():
        pltpu.make_async_copy(
            v_hbm.at[pl.ds((row_base + c * CH) * D, CH * D)],
            ring.at[pl.ds(slot * CH * D, CH * D)], sems.at[slot]).wait()

      @pl.when(part)
      def _():
        pltpu.make_async_copy(
            v_hbm.at[pl.ds((row_base + c * CH) * D, (CH - G) * D)],
            ring.at[pl.ds(slot * CH * D, (CH - G) * D)], sems.at[slot]).wait()

    for p in range(R - 1):
      issue(jnp.int32(p), p)

    idx_main_cp = pltpu.make_async_copy(
        b_hbm.at[pl.ds(row_base, IDX_MAIN)],
        idx_v.at[pl.ds(0, IDX_MAIN)], isems.at[0])
    idx_main_cp.start()
    idx_tail_cp = pltpu.make_async_copy(
        b_hbm.at[pl.ds(row_base + IDX_MAIN, G)],
        idx_v.at[pl.ds(IDX_MAIN, G)], isems.at[1])

    @pl.when(wid < GR)
    def _():
      idx_tail_cp.start()

    def fillz(i, _):
      for j in range(DJ):
        loc_s[pl.ds(i * D + j * L, L)] = zeros
      loc_c[pl.ds(i * L, L)] = zeros
      return 0

    lax.fori_loop(0, NUM_SEGMENTS, fillz, 0)

    idx_main_cp.wait()

    @pl.when(wid < GR)
    def _():
      idx_tail_cp.wait()

    def group_body(c, b):
      slot = lax.rem(c, R)
      valid = c * CH_G + b < ngroups
      idxv = idx_v[pl.ds(c * CH + b * G, G)]
      first = idxv[0]
      last = idxv[G - 1]
      base = slot * (CH * D) + b * (G * D)

      @pl.when(valid & (first == last))
      def _():
        acc = [ring[pl.ds(base + j * L, L)] for j in range(DJ)]
        for r in range(1, G):
          acc = [acc[j] + ring[pl.ds(base + r * D + j * L, L)]
                 for j in range(DJ)]
        sbase = first * D
        for j in range(DJ):
          loc_s[pl.ds(sbase + j * L, L)] = (
              loc_s[pl.ds(sbase + j * L, L)] + acc[j])
        loc_c[pl.ds(first * L, L)] = loc_c[pl.ds(first * L, L)] + sixteens

      @pl.when(valid & (first != last))
      def _():
        for r in range(G):
          s_r = idxv[r]
          rbase = s_r * D
          for j in range(DJ):
            loc_s[pl.ds(rbase + j * L, L)] = (
                loc_s[pl.ds(rbase + j * L, L)]
                + ring[pl.ds(base + r * D + j * L, L)])
          loc_c[pl.ds(s_r * L, L)] = loc_c[pl.ds(s_r * L, L)] + ones

      return c

    def chunk_body(c, _):
      wait_chunk(c, lax.rem(c, R))
      issue(c + (R - 1), lax.rem(c + (R - 1), R))
      for b in range(CH_G):
        group_body(c, b)
      return 0

    lax.fori_loop(0, NCH, chunk_body, 0)

    out_s = pltpu.make_async_copy(
        loc_s, psum_hbm.at[pl.ds(wid * SUMW, SUMW)], isems.at[0])
    out_c = pltpu.make_async_copy(loc_c, pcnt_hbm.at[wid], isems.at[1])
    out_s.start()
    out_c.start()
    out_s.wait()
    out_c.wait()

  return k(v_flat, batch)


def _combine_body(u_ref, ps_ref, pc_ref, o_ref):
  s = jnp.sum(ps_ref[...], axis=0)
  c = jnp.sum(pc_ref[...], axis=0)
  cnt = jnp.maximum(c[:, 0:1], 1.0)
  o_ref[...] = u_ref[...] + s / cnt


SEG_BLK = 128


def kernel(u, v, batch):
  batch = batch.astype(jnp.int32)
  psum, pcnt = _sc_partial(v.reshape(-1), batch)
  psum = psum.reshape(NW, NUM_SEGMENTS, D)
  pcnt = pcnt.reshape(NW, NUM_SEGMENTS, L)
  nblk = NUM_SEGMENTS // SEG_BLK
  return pl.pallas_call(
      _combine_body,
      grid=(nblk,),
      in_specs=[
          pl.BlockSpec((SEG_BLK, D), lambda i: (i, 0)),
          pl.BlockSpec((NW, SEG_BLK, D), lambda i: (0, i, 0)),
          pl.BlockSpec((NW, SEG_BLK, L), lambda i: (0, i, 0)),
      ],
      out_specs=pl.BlockSpec((SEG_BLK, D), lambda i: (i, 0)),
      out_shape=jax.ShapeDtypeStruct((NUM_SEGMENTS, D), jnp.float32),
  )(u, psum, pcnt)


# speedup vs baseline: 1.4146x; 1.4146x over previous
import functools

import jax
import jax.numpy as jnp
from jax import lax
from jax.experimental import pallas as pl
from jax.experimental.pallas import tpu as pltpu
from jax.experimental.pallas import tpu_sc as plsc

NUM_NODES = 100000
NUM_SEGMENTS = 512
D = 128

NC = 2
NS = 16
NW = NC * NS
L = 16
DJ = D // L

G = 16
NGROUPS = NUM_NODES // G
GQ, GR = divmod(NGROUPS, NW)
MAX_GROUPS = GQ + 1
IDX_MAIN = GQ * G
IDX_PAD = MAX_GROUPS * G

R = 3
CH = 64
CH_G = CH // G
NCH = (MAX_GROUPS + CH_G - 1) // CH_G
SUMW = NUM_SEGMENTS * D
CNTW = NUM_SEGMENTS * L


def _sc_partial(v_flat, batch):
  mesh = plsc.VectorSubcoreMesh(core_axis_name="c", subcore_axis_name="s")

  @functools.partial(
      pl.kernel,
      out_type=[
          jax.ShapeDtypeStruct((NW * SUMW,), jnp.float32),
          jax.ShapeDtypeStruct((NW, CNTW), jnp.float32),
      ],
      mesh=mesh,
      scratch_types=[
          pltpu.VMEM((IDX_PAD,), jnp.int32),
          pltpu.VMEM((R * CH * D,), jnp.float32),
          pltpu.VMEM((SUMW,), jnp.float32),
          pltpu.VMEM((CNTW,), jnp.float32),
          pltpu.SemaphoreType.DMA((R,)),
          pltpu.SemaphoreType.DMA((2,)),
      ],
  )
  def k(v_hbm, b_hbm, psum_hbm, pcnt_hbm, idx_v, ring, loc_s, loc_c, sems,
        isems):
    cid = lax.axis_index("c")
    sid = lax.axis_index("s")
    wid = sid * NC + cid

    base_group = wid * GQ + jnp.minimum(wid, GR)
    ngroups = GQ + jnp.where(wid < GR, 1, 0)
    row_base = base_group * G
    nrows = ngroups * G

    zeros = jnp.zeros((L,), jnp.float32)
    ones = jnp.ones((L,), jnp.float32)
    sixteens = jnp.full((L,), 16.0)

    def issue(c, slot):
      full = (c + 1) * CH <= nrows
      part = nrows - c * CH == CH - G

      @pl.when(full)
      def _():
        pltpu.async_copy(
            v_hbm.at[pl.ds((row_base + c * CH) * D, CH * D)],
            ring.at[pl.ds(slot * CH * D, CH * D)], sems.at[slot])

      @pl.when(part)
      def _():
        pltpu.async_copy(
            v_hbm.at[pl.ds((row_base + c * CH) * D, (CH - G) * D)],
            ring.at[pl.ds(slot * CH * D, (CH - G) * D)], sems.at[slot])

    def wait_chunk(c, slot):
      full = (c + 1) * CH <= nrows
      part = nrows - c * CH == CH - G

      @pl.when(full)
      def _():
        pltpu.make_async_copy(
            v_hbm.at[pl.ds((row_base + c * CH) * D, CH * D)],
            ring.at[pl.ds(slot * CH * D, CH * D)], sems.at[slot]).wait()

      @pl.when(part)
      def _():
        pltpu.make_async_copy(
            v_hbm.at[pl.ds((row_base + c * CH) * D, (CH - G) * D)],
            ring.at[pl.ds(slot * CH * D, (CH - G) * D)], sems.at[slot]).wait()

    for p in range(R - 1):
      issue(jnp.int32(p), p)

    idx_main_cp = pltpu.make_async_copy(
        b_hbm.at[pl.ds(row_base, IDX_MAIN)],
        idx_v.at[pl.ds(0, IDX_MAIN)], isems.at[0])
    idx_main_cp.start()
    idx_tail_cp = pltpu.make_async_copy(
        b_hbm.at[pl.ds(row_base + IDX_MAIN, G)],
        idx_v.at[pl.ds(IDX_MAIN, G)], isems.at[1])

    @pl.when(wid < GR)
    def _():
      idx_tail_cp.start()

    def fillz(i, _):
      for j in range(DJ):
        loc_s[pl.ds(i * D + j * L, L)] = zeros
      loc_c[pl.ds(i * L, L)] = zeros
      return 0

    lax.fori_loop(0, NUM_SEGMENTS, fillz, 0)

    idx_main_cp.wait()

    @pl.when(wid < GR)
    def _():
      idx_tail_cp.wait()

    def group_body(b, c):
      slot = lax.rem(c, R)
      valid = c * CH_G + b < ngroups
      idxv = idx_v[pl.ds(c * CH + b * G, G)]
      first = idxv[0]
      last = idxv[G - 1]
      base = slot * (CH * D) + b * (G * D)

      @pl.when(valid & (first == last))
      def _():
        acc = [ring[pl.ds(base + j * L, L)] for j in range(DJ)]
        for r in range(1, G):
          acc = [acc[j] + ring[pl.ds(base + r * D + j * L, L)]
                 for j in range(DJ)]
        sbase = first * D
        for j in range(DJ):
          loc_s[pl.ds(sbase + j * L, L)] = (
              loc_s[pl.ds(sbase + j * L, L)] + acc[j])
        loc_c[pl.ds(first * L, L)] = loc_c[pl.ds(first * L, L)] + sixteens

      @pl.when(valid & (first != last))
      def _():
        for r in range(G):
          s_r = idxv[r]
          rbase = s_r * D
          for j in range(DJ):
            loc_s[pl.ds(rbase + j * L, L)] = (
                loc_s[pl.ds(rbase + j * L, L)]
                + ring[pl.ds(base + r * D + j * L, L)])
          loc_c[pl.ds(s_r * L, L)] = loc_c[pl.ds(s_r * L, L)] + ones

      return c

    def chunk_body(c, _):
      wait_chunk(c, lax.rem(c, R))
      issue(c + (R - 1), lax.rem(c + (R - 1), R))
      lax.fori_loop(0, CH_G, group_body, c)
      return 0

    lax.fori_loop(0, NCH, chunk_body, 0)

    out_s = pltpu.make_async_copy(
        loc_s, psum_hbm.at[pl.ds(wid * SUMW, SUMW)], isems.at[0])
    out_c = pltpu.make_async_copy(loc_c, pcnt_hbm.at[wid], isems.at[1])
    out_s.start()
    out_c.start()
    out_s.wait()
    out_c.wait()

  return k(v_flat, batch)


def _combine_body(u_ref, ps_ref, pc_ref, o_ref):
  s = jnp.sum(ps_ref[...], axis=0)
  c = jnp.sum(pc_ref[...], axis=0)
  cnt = jnp.maximum(c[:, 0:1], 1.0)
  o_ref[...] = u_ref[...] + s / cnt


SEG_BLK = 128


def kernel(u, v, batch):
  batch = batch.astype(jnp.int32)
  psum, pcnt = _sc_partial(v.reshape(-1), batch)
  psum = psum.reshape(NW, NUM_SEGMENTS, D)
  pcnt = pcnt.reshape(NW, NUM_SEGMENTS, L)
  nblk = NUM_SEGMENTS // SEG_BLK
  return pl.pallas_call(
      _combine_body,
      grid=(nblk,),
      in_specs=[
          pl.BlockSpec((SEG_BLK, D), lambda i: (i, 0)),
          pl.BlockSpec((NW, SEG_BLK, D), lambda i: (0, i, 0)),
          pl.BlockSpec((NW, SEG_BLK, L), lambda i: (0, i, 0)),
      ],
      out_specs=pl.BlockSpec((SEG_BLK, D), lambda i: (i, 0)),
      out_shape=jax.ShapeDtypeStruct((NUM_SEGMENTS, D), jnp.float32),
  )(u, psum, pcnt)


# speedup vs baseline: 1.4176x; 1.0021x over previous
import functools

import jax
import jax.numpy as jnp
from jax import lax
from jax.experimental import pallas as pl
from jax.experimental.pallas import tpu as pltpu
from jax.experimental.pallas import tpu_sc as plsc

NUM_NODES = 100000
NUM_SEGMENTS = 512
D = 128

NC = 2
NS = 16
NW = NC * NS
L = 16
DJ = D // L

G = 16
NGROUPS = NUM_NODES // G
GQ, GR = divmod(NGROUPS, NW)
MAX_GROUPS = GQ + 1
IDX_MAIN = GQ * G
IDX_PAD = MAX_GROUPS * G

R = 4
CH = 64
CH_G = CH // G
NCH = (MAX_GROUPS + CH_G - 1) // CH_G
SUMW = NUM_SEGMENTS * D
CNTW = NUM_SEGMENTS * L


def _sc_partial(v_flat, batch):
  mesh = plsc.VectorSubcoreMesh(core_axis_name="c", subcore_axis_name="s")

  @functools.partial(
      pl.kernel,
      out_type=[
          jax.ShapeDtypeStruct((NW * SUMW,), jnp.float32),
          jax.ShapeDtypeStruct((NW, CNTW), jnp.float32),
      ],
      mesh=mesh,
      scratch_types=[
          pltpu.VMEM((IDX_PAD,), jnp.int32),
          pltpu.VMEM((R * CH * D,), jnp.float32),
          pltpu.VMEM((SUMW,), jnp.float32),
          pltpu.VMEM((CNTW,), jnp.float32),
          pltpu.SemaphoreType.DMA((R,)),
          pltpu.SemaphoreType.DMA((2,)),
      ],
  )
  def k(v_hbm, b_hbm, psum_hbm, pcnt_hbm, idx_v, ring, loc_s, loc_c, sems,
        isems):
    cid = lax.axis_index("c")
    sid = lax.axis_index("s")
    wid = sid * NC + cid

    base_group = wid * GQ + jnp.minimum(wid, GR)
    ngroups = GQ + jnp.where(wid < GR, 1, 0)
    row_base = base_group * G
    nrows = ngroups * G

    zeros = jnp.zeros((L,), jnp.float32)
    ones = jnp.ones((L,), jnp.float32)
    sixteens = jnp.full((L,), 16.0)

    def issue(c, slot):
      full = (c + 1) * CH <= nrows
      part = nrows - c * CH == CH - G

      @pl.when(full)
      def _():
        pltpu.async_copy(
            v_hbm.at[pl.ds((row_base + c * CH) * D, CH * D)],
            ring.at[pl.ds(slot * CH * D, CH * D)], sems.at[slot])

      @pl.when(part)
      def _():
        pltpu.async_copy(
            v_hbm.at[pl.ds((row_base + c * CH) * D, (CH - G) * D)],
            ring.at[pl.ds(slot * CH * D, (CH - G) * D)], sems.at[slot])

    def wait_chunk(c, slot):
      full = (c + 1) * CH <= nrows
      part = nrows - c * CH == CH - G

      @pl.when(full)
      def _():
        pltpu.make_async_copy(
            v_hbm.at[pl.ds((row_base + c * CH) * D, CH * D)],
            ring.at[pl.ds(slot * CH * D, CH * D)], sems.at[slot]).wait()

      @pl.when(part)
      def _():
        pltpu.make_async_copy(
            v_hbm.at[pl.ds((row_base + c * CH) * D, (CH - G) * D)],
            ring.at[pl.ds(slot * CH * D, (CH - G) * D)], sems.at[slot]).wait()

    for p in range(R - 1):
      issue(jnp.int32(p), p)

    idx_main_cp = pltpu.make_async_copy(
        b_hbm.at[pl.ds(row_base, IDX_MAIN)],
        idx_v.at[pl.ds(0, IDX_MAIN)], isems.at[0])
    idx_main_cp.start()
    idx_tail_cp = pltpu.make_async_copy(
        b_hbm.at[pl.ds(row_base + IDX_MAIN, G)],
        idx_v.at[pl.ds(IDX_MAIN, G)], isems.at[1])

    @pl.when(wid < GR)
    def _():
      idx_tail_cp.start()

    def fillz(i, _):
      for j in range(DJ):
        loc_s[pl.ds(i * D + j * L, L)] = zeros
      loc_c[pl.ds(i * L, L)] = zeros
      return 0

    lax.fori_loop(0, NUM_SEGMENTS, fillz, 0)

    idx_main_cp.wait()

    @pl.when(wid < GR)
    def _():
      idx_tail_cp.wait()

    def group_body(b, c):
      slot = lax.rem(c, R)
      valid = c * CH_G + b < ngroups
      idxv = idx_v[pl.ds(c * CH + b * G, G)]
      first = idxv[0]
      last = idxv[G - 1]
      base = slot * (CH * D) + b * (G * D)

      @pl.when(valid & (first == last))
      def _():
        acc = [ring[pl.ds(base + j * L, L)] for j in range(DJ)]
        for r in range(1, G):
          acc = [acc[j] + ring[pl.ds(base + r * D + j * L, L)]
                 for j in range(DJ)]
        sbase = first * D
        for j in range(DJ):
          loc_s[pl.ds(sbase + j * L, L)] = (
              loc_s[pl.ds(sbase + j * L, L)] + acc[j])
        loc_c[pl.ds(first * L, L)] = loc_c[pl.ds(first * L, L)] + sixteens

      @pl.when(valid & (first != last))
      def _():
        for r in range(G):
          s_r = idxv[r]
          rbase = s_r * D
          for j in range(DJ):
            loc_s[pl.ds(rbase + j * L, L)] = (
                loc_s[pl.ds(rbase + j * L, L)]
                + ring[pl.ds(base + r * D + j * L, L)])
          loc_c[pl.ds(s_r * L, L)] = loc_c[pl.ds(s_r * L, L)] + ones

      return c

    def chunk_body(c, _):
      wait_chunk(c, lax.rem(c, R))
      issue(c + (R - 1), lax.rem(c + (R - 1), R))
      lax.fori_loop(0, CH_G, group_body, c)
      return 0

    lax.fori_loop(0, NCH, chunk_body, 0)

    out_s = pltpu.make_async_copy(
        loc_s, psum_hbm.at[pl.ds(wid * SUMW, SUMW)], isems.at[0])
    out_c = pltpu.make_async_copy(loc_c, pcnt_hbm.at[wid], isems.at[1])
    out_s.start()
    out_c.start()
    out_s.wait()
    out_c.wait()

  return k(v_flat, batch)


def _combine_body(u_ref, ps_ref, pc_ref, o_ref):
  s = jnp.sum(ps_ref[...], axis=0)
  c = jnp.sum(pc_ref[...], axis=0)
  cnt = jnp.maximum(c[:, 0:1], 1.0)
  o_ref[...] = u_ref[...] + s / cnt


SEG_BLK = 128


def kernel(u, v, batch):
  batch = batch.astype(jnp.int32)
  psum, pcnt = _sc_partial(v.reshape(-1), batch)
  psum = psum.reshape(NW, NUM_SEGMENTS, D)
  pcnt = pcnt.reshape(NW, NUM_SEGMENTS, L)
  nblk = NUM_SEGMENTS // SEG_BLK
  return pl.pallas_call(
      _combine_body,
      grid=(nblk,),
      in_specs=[
          pl.BlockSpec((SEG_BLK, D), lambda i: (i, 0)),
          pl.BlockSpec((NW, SEG_BLK, D), lambda i: (0, i, 0)),
          pl.BlockSpec((NW, SEG_BLK, L), lambda i: (0, i, 0)),
      ],
      out_specs=pl.BlockSpec((SEG_BLK, D), lambda i: (i, 0)),
      out_shape=jax.ShapeDtypeStruct((NUM_SEGMENTS, D), jnp.float32),
  )(u, psum, pcnt)


# speedup vs baseline: 1.4193x; 1.0012x over previous
import functools

import jax
import jax.numpy as jnp
from jax import lax
from jax.experimental import pallas as pl
from jax.experimental.pallas import tpu as pltpu
from jax.experimental.pallas import tpu_sc as plsc

NUM_NODES = 100000
NUM_SEGMENTS = 512
D = 128

NC = 2
NS = 16
NW = NC * NS
L = 16
DJ = D // L

G = 16
NGROUPS = NUM_NODES // G
GQ, GR = divmod(NGROUPS, NW)
MAX_GROUPS = GQ + 1
IDX_MAIN = GQ * G
IDX_PAD = MAX_GROUPS * G

R = 3
CH = 128
CH_G = CH // G
NCH = (MAX_GROUPS + CH_G - 1) // CH_G
PART_SIZES = sorted({sz for sz in (GQ * G % CH, (GQ + 1) * G % CH) if sz})
SUMW = NUM_SEGMENTS * D
CNTW = NUM_SEGMENTS * L


def _sc_partial(v_flat, batch):
  mesh = plsc.VectorSubcoreMesh(core_axis_name="c", subcore_axis_name="s")

  @functools.partial(
      pl.kernel,
      out_type=[
          jax.ShapeDtypeStruct((NW * SUMW,), jnp.float32),
          jax.ShapeDtypeStruct((NW, CNTW), jnp.float32),
      ],
      mesh=mesh,
      scratch_types=[
          pltpu.VMEM((IDX_PAD,), jnp.int32),
          pltpu.VMEM((R * CH * D,), jnp.float32),
          pltpu.VMEM((SUMW,), jnp.float32),
          pltpu.VMEM((CNTW,), jnp.float32),
          pltpu.SemaphoreType.DMA((R,)),
          pltpu.SemaphoreType.DMA((2,)),
      ],
  )
  def k(v_hbm, b_hbm, psum_hbm, pcnt_hbm, idx_v, ring, loc_s, loc_c, sems,
        isems):
    cid = lax.axis_index("c")
    sid = lax.axis_index("s")
    wid = sid * NC + cid

    base_group = wid * GQ + jnp.minimum(wid, GR)
    ngroups = GQ + jnp.where(wid < GR, 1, 0)
    row_base = base_group * G
    nrows = ngroups * G

    zeros = jnp.zeros((L,), jnp.float32)
    ones = jnp.ones((L,), jnp.float32)
    sixteens = jnp.full((L,), 16.0)

    def issue(c, slot):
      @pl.when((c + 1) * CH <= nrows)
      def _():
        pltpu.async_copy(
            v_hbm.at[pl.ds((row_base + c * CH) * D, CH * D)],
            ring.at[pl.ds(slot * CH * D, CH * D)], sems.at[slot])

      for sz in PART_SIZES:
        @pl.when(nrows - c * CH == sz)
        def _(sz=sz):
          pltpu.async_copy(
              v_hbm.at[pl.ds((row_base + c * CH) * D, sz * D)],
              ring.at[pl.ds(slot * CH * D, sz * D)], sems.at[slot])

    def wait_chunk(c, slot):
      @pl.when((c + 1) * CH <= nrows)
      def _():
        pltpu.make_async_copy(
            v_hbm.at[pl.ds((row_base + c * CH) * D, CH * D)],
            ring.at[pl.ds(slot * CH * D, CH * D)], sems.at[slot]).wait()

      for sz in PART_SIZES:
        @pl.when(nrows - c * CH == sz)
        def _(sz=sz):
          pltpu.make_async_copy(
              v_hbm.at[pl.ds((row_base + c * CH) * D, sz * D)],
              ring.at[pl.ds(slot * CH * D, sz * D)], sems.at[slot]).wait()

    for p in range(R - 1):
      issue(jnp.int32(p), p)

    idx_main_cp = pltpu.make_async_copy(
        b_hbm.at[pl.ds(row_base, IDX_MAIN)],
        idx_v.at[pl.ds(0, IDX_MAIN)], isems.at[0])
    idx_main_cp.start()
    idx_tail_cp = pltpu.make_async_copy(
        b_hbm.at[pl.ds(row_base + IDX_MAIN, G)],
        idx_v.at[pl.ds(IDX_MAIN, G)], isems.at[1])

    @pl.when(wid < GR)
    def _():
      idx_tail_cp.start()

    def fillz(i, _):
      for j in range(DJ):
        loc_s[pl.ds(i * D + j * L, L)] = zeros
      loc_c[pl.ds(i * L, L)] = zeros
      return 0

    lax.fori_loop(0, NUM_SEGMENTS, fillz, 0)

    idx_main_cp.wait()

    @pl.when(wid < GR)
    def _():
      idx_tail_cp.wait()

    def group_body(b, c):
      slot = lax.rem(c, R)
      valid = c * CH_G + b < ngroups
      idxv = idx_v[pl.ds(c * CH + b * G, G)]
      first = idxv[0]
      last = idxv[G - 1]
      base = slot * (CH * D) + b * (G * D)

      @pl.when(valid & (first == last))
      def _():
        acc = [ring[pl.ds(base + j * L, L)] for j in range(DJ)]
        for r in range(1, G):
          acc = [acc[j] + ring[pl.ds(base + r * D + j * L, L)]
                 for j in range(DJ)]
        sbase = first * D
        for j in range(DJ):
          loc_s[pl.ds(sbase + j * L, L)] = (
              loc_s[pl.ds(sbase + j * L, L)] + acc[j])
        loc_c[pl.ds(first * L, L)] = loc_c[pl.ds(first * L, L)] + sixteens

      @pl.when(valid & (first != last))
      def _():
        for r in range(G):
          s_r = idxv[r]
          rbase = s_r * D
          for j in range(DJ):
            loc_s[pl.ds(rbase + j * L, L)] = (
                loc_s[pl.ds(rbase + j * L, L)]
                + ring[pl.ds(base + r * D + j * L, L)])
          loc_c[pl.ds(s_r * L, L)] = loc_c[pl.ds(s_r * L, L)] + ones

      return c

    def chunk_body(c, _):
      wait_chunk(c, lax.rem(c, R))
      issue(c + (R - 1), lax.rem(c + (R - 1), R))
      lax.fori_loop(0, CH_G, group_body, c)
      return 0

    lax.fori_loop(0, NCH, chunk_body, 0)

    out_s = pltpu.make_async_copy(
        loc_s, psum_hbm.at[pl.ds(wid * SUMW, SUMW)], isems.at[0])
    out_c = pltpu.make_async_copy(loc_c, pcnt_hbm.at[wid], isems.at[1])
    out_s.start()
    out_c.start()
    out_s.wait()
    out_c.wait()

  return k(v_flat, batch)


def _combine_body(u_ref, ps_ref, pc_ref, o_ref):
  s = jnp.sum(ps_ref[...], axis=0)
  c = jnp.sum(pc_ref[...], axis=0)
  cnt = jnp.maximum(c[:, 0:1], 1.0)
  o_ref[...] = u_ref[...] + s / cnt


SEG_BLK = 128


def kernel(u, v, batch):
  batch = batch.astype(jnp.int32)
  psum, pcnt = _sc_partial(v.reshape(-1), batch)
  psum = psum.reshape(NW, NUM_SEGMENTS, D)
  pcnt = pcnt.reshape(NW, NUM_SEGMENTS, L)
  nblk = NUM_SEGMENTS // SEG_BLK
  return pl.pallas_call(
      _combine_body,
      grid=(nblk,),
      in_specs=[
          pl.BlockSpec((SEG_BLK, D), lambda i: (i, 0)),
          pl.BlockSpec((NW, SEG_BLK, D), lambda i: (0, i, 0)),
          pl.BlockSpec((NW, SEG_BLK, L), lambda i: (0, i, 0)),
      ],
      out_specs=pl.BlockSpec((SEG_BLK, D), lambda i: (i, 0)),
      out_shape=jax.ShapeDtypeStruct((NUM_SEGMENTS, D), jnp.float32),
  )(u, psum, pcnt)
